# baseline (speedup 1.0000x reference)
import jax
import jax.numpy as jnp
from jax import lax
from jax.experimental import pallas as pl
from jax.experimental.pallas import tpu as pltpu
from jax.experimental.pallas import tpu_sc as plsc

_IPW = 8
_L = 16


def _sc_body(user_hbm, items_hbm, q_hbm, emb_hbm, out_hbm,
             user_v, idx_v, uidx_v, q_v, qf_v, emb_v, emb_f, ret_v,
             sem_q, sem_e):
    n_workers = 200 // _IPW
    skills = q_hbm.shape[1]
    k_hidden = 64
    nkc = k_hidden // _L
    nsc = skills // _L
    wid = lax.axis_index("s") * 2 + lax.axis_index("c")
    iota = jnp.arange(_L, dtype=jnp.int32)

    @pl.when(wid < n_workers)
    def _():
        base = wid * _IPW
        pltpu.sync_copy(user_hbm, user_v)
        pltpu.sync_copy(items_hbm.at[pl.ds(base, _IPW)], idx_v)
        uv = user_v[...] * k_hidden
        for c in range(k_hidden // _L):
            uidx_v[pl.ds(c * _L, _L)] = uv + (iota + c * _L)
        emb_copy = pltpu.make_async_copy(emb_hbm.at[uidx_v], emb_v, sem_e)
        emb_copy.start()
        q_copy = pltpu.make_async_copy(q_hbm.at[idx_v], q_v, sem_q)
        q_copy.start()
        q_copy.wait()
        for i in range(_IPW):
            for c in range(nsc):
                qf_v[pl.ds(i * skills + c * _L, _L)] = (
                    q_v[i, pl.ds(c * _L, _L)].astype(jnp.float32))
        emb_copy.wait()
        for r in range(k_hidden):
            for c in range(nsc):
                emb_f[pl.ds(r * skills + c * _L, _L)] = (
                    emb_v[r, pl.ds(c * _L, _L)])

        kidx = [(iota + c * _L) * skills for c in range(nkc)]
        ibase = [jnp.full((_L,), i * skills, jnp.int32) for i in range(_IPW)]

        def sbody(s, accs):
            sv = jnp.full((_L,), s, jnp.int32)
            evs = [plsc.load_gather(emb_f, [kidx[c] + sv])
                   for c in range(nkc)]
            new = []
            for i in range(_IPW):
                qb = plsc.load_gather(qf_v, [ibase[i] + sv])
                new.extend(accs[i * nkc + c] + qb * evs[c]
                           for c in range(nkc))
            return tuple(new)

        accs0 = tuple(jnp.zeros((_L,), jnp.float32)
                      for _ in range(_IPW * nkc))
        accs = lax.fori_loop(0, skills, sbody, accs0)
        for i in range(_IPW):
            for c in range(nkc):
                ret_v[i, pl.ds(c * _L, _L)] = accs[i * nkc + c]
        pltpu.sync_copy(ret_v, out_hbm.at[pl.ds(base, _IPW)])


def kernel(user, Q_matrix, items, skill_embedding):
    seq_len = items.shape[0]
    n_items, skill_num = Q_matrix.shape
    k_hidden = skill_embedding.shape[2]
    n_users = skill_embedding.shape[0]
    user_arr = jnp.full((_L,), user, jnp.int32)
    emb_t = jnp.swapaxes(skill_embedding, 1, 2).reshape(
        n_users * k_hidden, skill_num)

    mesh = plsc.VectorSubcoreMesh(core_axis_name="c", subcore_axis_name="s")
    run = pl.kernel(
        _sc_body,
        out_type=jax.ShapeDtypeStruct((seq_len, k_hidden), jnp.float32),
        mesh=mesh,
        compiler_params=pltpu.CompilerParams(needs_layout_passes=False),
        scratch_types=[
            pltpu.VMEM((_L,), jnp.int32),
            pltpu.VMEM((_IPW,), jnp.int32),
            pltpu.VMEM((k_hidden,), jnp.int32),
            pltpu.VMEM((_IPW, skill_num), jnp.int32),
            pltpu.VMEM((_IPW * skill_num,), jnp.float32),
            pltpu.VMEM((k_hidden, skill_num), jnp.float32),
            pltpu.VMEM((k_hidden * skill_num,), jnp.float32),
            pltpu.VMEM((_IPW, k_hidden), jnp.float32),
            pltpu.SemaphoreType.DMA,
            pltpu.SemaphoreType.DMA,
        ],
    )
    return run(user_arr, items.astype(jnp.int32), Q_matrix, emb_t)
